# baseline (speedup 1.0000x reference)
import jax
import jax.numpy as jnp
from jax.experimental import pallas as pl
from jax.experimental.pallas import tpu as pltpu

BS, T, C_IN = 4, 8, 256
K = 17
NODE_DIM = 1024
HID = 256
PRED_PAD = 128

_PREC = None


def _fused_body(x0_ref, x1_ref, x2_ref, x3_ref, A_ref, Wc_ref, bc_ref,
                Wg_ref, bg_ref, Wzrn_ref, bzrn_ref, Uzrn_ref, Wp_ref, bp_ref,
                feat_ref, pred_ref, h_scr):
    x_refs = (x0_ref, x1_ref, x2_ref, x3_ref)
    j = pl.program_id(0)

    @pl.when(j == 0)
    def _init():
        h_scr[...] = jnp.zeros_like(h_scr)

    AW = jnp.dot(A_ref[:], Wc_ref[:], precision=_PREC,
                 preferred_element_type=jnp.float32).astype(jnp.bfloat16)
    ab = jnp.sum(A_ref[:] * bc_ref[:], axis=1, keepdims=True)

    for b in range(BS):
        g = jnp.dot(AW, x_refs[b][0], precision=_PREC,
                    preferred_element_type=jnp.float32) + ab
        gw = jnp.maximum(
            jnp.dot(g.astype(jnp.bfloat16), Wg_ref[:], precision=_PREC,
                    preferred_element_type=jnp.float32) + bg_ref[:], 0.0)
        xp = jnp.dot(gw, Wzrn_ref[:], precision=_PREC,
                     preferred_element_type=jnp.float32) + bzrn_ref[:]
        h = h_scr[b]
        hu = jnp.dot(h, Uzrn_ref[:], precision=_PREC,
                     preferred_element_type=jnp.float32)
        z = jax.nn.sigmoid(xp[:, :HID] + hu[:, :HID])
        r = jax.nn.sigmoid(xp[:, HID:2 * HID] + hu[:, HID:2 * HID])
        n = jnp.tanh(xp[:, 2 * HID:] + r * hu[:, 2 * HID:])
        hn = h + z * (n - h)
        h_scr[b] = hn
        feat_ref[0, b * K:(b + 1) * K] = hn
        pred_ref[0, b * K:(b + 1) * K] = jnp.dot(
            hn, Wp_ref[:], precision=_PREC,
            preferred_element_type=jnp.float32) + bp_ref[:]


def kernel(x, A, Wconv, bconv, Wg, bg, Wz, Uz, bz, Wr, Ur, br,
           Wn, Un, bn, Wp, bp):
    b, t, c, h, w = x.shape
    xf = x.reshape(b * t, c, h * w).astype(jnp.bfloat16)
    Wgb = Wg.astype(jnp.bfloat16)

    Wzrn = jnp.concatenate([Wz, Wr, Wn], axis=1)
    bzrn = jnp.concatenate([bz, br, bn]).reshape(1, 3 * HID)
    Uzrn = jnp.concatenate([Uz, Ur, Un], axis=1)
    Wp_pad = jnp.zeros((HID, PRED_PAD), jnp.float32).at[:, :2].set(Wp)
    bp_pad = jnp.zeros((1, PRED_PAD), jnp.float32).at[:, :2].set(bp)

    def _xspec(bb):
        return pl.BlockSpec((1, c, h * w), lambda j, bb=bb: (bb * t + j, 0, 0))

    feat_t, pred_t = pl.pallas_call(
        _fused_body,
        grid=(t,),
        in_specs=[
            _xspec(0), _xspec(1), _xspec(2), _xspec(3),
            pl.BlockSpec((K, K), lambda j: (0, 0)),
            pl.BlockSpec((K, c), lambda j: (0, 0)),
            pl.BlockSpec((1, K), lambda j: (0, 0)),
            pl.BlockSpec((h * w, HID), lambda j: (0, 0)),
            pl.BlockSpec((1, HID), lambda j: (0, 0)),
            pl.BlockSpec((HID, 3 * HID), lambda j: (0, 0)),
            pl.BlockSpec((1, 3 * HID), lambda j: (0, 0)),
            pl.BlockSpec((HID, 3 * HID), lambda j: (0, 0)),
            pl.BlockSpec((HID, PRED_PAD), lambda j: (0, 0)),
            pl.BlockSpec((1, PRED_PAD), lambda j: (0, 0)),
        ],
        out_specs=[
            pl.BlockSpec((1, BS * K, HID), lambda j: (j, 0, 0)),
            pl.BlockSpec((1, BS * K, PRED_PAD), lambda j: (j, 0, 0)),
        ],
        out_shape=[
            jax.ShapeDtypeStruct((t, BS * K, HID), jnp.float32),
            jax.ShapeDtypeStruct((t, BS * K, PRED_PAD), jnp.float32),
        ],
        scratch_shapes=[pltpu.VMEM((BS, K, HID), jnp.float32)],
        compiler_params=pltpu.CompilerParams(
            dimension_semantics=("arbitrary",)),
    )(xf, xf, xf, xf, A, Wconv, bconv.reshape(1, K), Wgb, bg.reshape(1, HID),
      Wzrn, bzrn, Uzrn, Wp_pad, bp_pad)

    feat = feat_t.reshape(t, b, K, HID).transpose(1, 0, 2, 3)
    pred = pred_t[..., :2].reshape(t, b, K, 2).transpose(1, 0, 2, 3)
    return pred, feat


# speedup vs baseline: 1.0642x; 1.0642x over previous
import jax
import jax.numpy as jnp
from jax.experimental import pallas as pl
from jax.experimental.pallas import tpu as pltpu

BS, T, C_IN = 4, 8, 256
K = 17
NODE_DIM = 1024
HID = 256
PRED_PAD = 128

_PREC = None


def _fused_body(x0_ref, x1_ref, x2_ref, x3_ref, A_ref, Wc_ref, bc_ref,
                Wg_ref, bg_ref, Wzrn_ref, bzrn_ref, Uzrn_ref, Wp_ref, bp_ref,
                feat_ref, pred_ref, h_scr):
    x_refs = (x0_ref, x1_ref, x2_ref, x3_ref)
    j = pl.program_id(0)

    @pl.when(j == 0)
    def _init():
        h_scr[...] = jnp.zeros_like(h_scr)

    AW = jnp.dot(A_ref[:], Wc_ref[:], precision=_PREC,
                 preferred_element_type=jnp.float32).astype(jnp.bfloat16)
    ab = jnp.sum(A_ref[:] * bc_ref[:], axis=1, keepdims=True)

    for b in range(BS):
        g = jnp.dot(AW, x_refs[b][0].astype(jnp.bfloat16), precision=_PREC,
                    preferred_element_type=jnp.float32) + ab
        gw = jnp.maximum(
            jnp.dot(g.astype(jnp.bfloat16), Wg_ref[:], precision=_PREC,
                    preferred_element_type=jnp.float32) + bg_ref[:], 0.0)
        xp = jnp.dot(gw, Wzrn_ref[:], precision=_PREC,
                     preferred_element_type=jnp.float32) + bzrn_ref[:]
        h = h_scr[b]
        hu = jnp.dot(h, Uzrn_ref[:], precision=_PREC,
                     preferred_element_type=jnp.float32)
        z = jax.nn.sigmoid(xp[:, :HID] + hu[:, :HID])
        r = jax.nn.sigmoid(xp[:, HID:2 * HID] + hu[:, HID:2 * HID])
        n = jnp.tanh(xp[:, 2 * HID:] + r * hu[:, 2 * HID:])
        hn = h + z * (n - h)
        h_scr[b] = hn
        feat_ref[0, b * K:(b + 1) * K] = hn
        pred_ref[0, b * K:(b + 1) * K] = jnp.dot(
            hn, Wp_ref[:], precision=_PREC,
            preferred_element_type=jnp.float32) + bp_ref[:]


def kernel(x, A, Wconv, bconv, Wg, bg, Wz, Uz, bz, Wr, Ur, br,
           Wn, Un, bn, Wp, bp):
    b, t, c, h, w = x.shape
    xf = x.reshape(b * t, c, h * w)
    Wgb = Wg.astype(jnp.bfloat16)

    Wzrn = jnp.concatenate([Wz, Wr, Wn], axis=1)
    bzrn = jnp.concatenate([bz, br, bn]).reshape(1, 3 * HID)
    Uzrn = jnp.concatenate([Uz, Ur, Un], axis=1)
    Wp_pad = jnp.zeros((HID, PRED_PAD), jnp.float32).at[:, :2].set(Wp)
    bp_pad = jnp.zeros((1, PRED_PAD), jnp.float32).at[:, :2].set(bp)

    def _xspec(bb):
        return pl.BlockSpec((1, c, h * w), lambda j, bb=bb: (bb * t + j, 0, 0))

    feat_t, pred_t = pl.pallas_call(
        _fused_body,
        grid=(t,),
        in_specs=[
            _xspec(0), _xspec(1), _xspec(2), _xspec(3),
            pl.BlockSpec((K, K), lambda j: (0, 0)),
            pl.BlockSpec((K, c), lambda j: (0, 0)),
            pl.BlockSpec((1, K), lambda j: (0, 0)),
            pl.BlockSpec((h * w, HID), lambda j: (0, 0)),
            pl.BlockSpec((1, HID), lambda j: (0, 0)),
            pl.BlockSpec((HID, 3 * HID), lambda j: (0, 0)),
            pl.BlockSpec((1, 3 * HID), lambda j: (0, 0)),
            pl.BlockSpec((HID, 3 * HID), lambda j: (0, 0)),
            pl.BlockSpec((HID, PRED_PAD), lambda j: (0, 0)),
            pl.BlockSpec((1, PRED_PAD), lambda j: (0, 0)),
        ],
        out_specs=[
            pl.BlockSpec((1, BS * K, HID), lambda j: (j, 0, 0)),
            pl.BlockSpec((1, BS * K, PRED_PAD), lambda j: (j, 0, 0)),
        ],
        out_shape=[
            jax.ShapeDtypeStruct((t, BS * K, HID), jnp.float32),
            jax.ShapeDtypeStruct((t, BS * K, PRED_PAD), jnp.float32),
        ],
        scratch_shapes=[pltpu.VMEM((BS, K, HID), jnp.float32)],
        compiler_params=pltpu.CompilerParams(
            dimension_semantics=("arbitrary",)),
    )(xf, xf, xf, xf, A, Wconv, bconv.reshape(1, K), Wgb, bg.reshape(1, HID),
      Wzrn, bzrn, Uzrn, Wp_pad, bp_pad)

    feat = feat_t.reshape(t, b, K, HID).transpose(1, 0, 2, 3)
    pred = pred_t[..., :2].reshape(t, b, K, 2).transpose(1, 0, 2, 3)
    return pred, feat


# speedup vs baseline: 1.2011x; 1.1287x over previous
import jax
import jax.numpy as jnp
from jax.experimental import pallas as pl
from jax.experimental.pallas import tpu as pltpu

BS, T, C_IN = 4, 8, 256
K = 17
NODE_DIM = 1024
HID = 256
PRED_PAD = 128

_PREC = None


def _fused_body(x0_ref, x1_ref, x2_ref, x3_ref, A_ref, Wc_ref, bc_ref,
                Wg_ref, bg_ref, Wzrn_ref, bzrn_ref, Uzrn_ref, Wp_ref, bp_ref,
                feat_ref, pred_ref, h_scr):
    x_refs = (x0_ref, x1_ref, x2_ref, x3_ref)
    j = pl.program_id(0)

    @pl.when(j == 0)
    def _init():
        h_scr[...] = jnp.zeros_like(h_scr)

    AW = jnp.dot(A_ref[:], Wc_ref[:], precision=_PREC,
                 preferred_element_type=jnp.float32).astype(jnp.bfloat16)
    ab = jnp.sum(A_ref[:] * bc_ref[:], axis=1, keepdims=True)

    hs = [h_scr[b] for b in range(BS)]
    hu = [jnp.dot(hs[b], Uzrn_ref[:], precision=_PREC,
                  preferred_element_type=jnp.float32) for b in range(BS)]
    g = [jnp.dot(AW, x_refs[b][0].astype(jnp.bfloat16), precision=_PREC,
                 preferred_element_type=jnp.float32) + ab for b in range(BS)]
    gw = [jnp.maximum(
        jnp.dot(g[b].astype(jnp.bfloat16), Wg_ref[:], precision=_PREC,
                preferred_element_type=jnp.float32) + bg_ref[:], 0.0)
        for b in range(BS)]
    xp = [jnp.dot(gw[b], Wzrn_ref[:], precision=_PREC,
                  preferred_element_type=jnp.float32) + bzrn_ref[:]
          for b in range(BS)]
    for b in range(BS):
        z = jax.nn.sigmoid(xp[b][:, :HID] + hu[b][:, :HID])
        r = jax.nn.sigmoid(xp[b][:, HID:2 * HID] + hu[b][:, HID:2 * HID])
        n = jnp.tanh(xp[b][:, 2 * HID:] + r * hu[b][:, 2 * HID:])
        hn = hs[b] + z * (n - hs[b])
        h_scr[b] = hn
        feat_ref[0, b * K:(b + 1) * K] = hn
        pred_ref[0, b * K:(b + 1) * K] = jnp.dot(
            hn, Wp_ref[:], precision=_PREC,
            preferred_element_type=jnp.float32) + bp_ref[:]


def kernel(x, A, Wconv, bconv, Wg, bg, Wz, Uz, bz, Wr, Ur, br,
           Wn, Un, bn, Wp, bp):
    b, t, c, h, w = x.shape
    xf = x.reshape(b * t, c, h * w)
    Wgb = Wg.astype(jnp.bfloat16)

    Wzrn = jnp.concatenate([Wz, Wr, Wn], axis=1)
    bzrn = jnp.concatenate([bz, br, bn]).reshape(1, 3 * HID)
    Uzrn = jnp.concatenate([Uz, Ur, Un], axis=1)
    Wp_pad = jnp.zeros((HID, PRED_PAD), jnp.float32).at[:, :2].set(Wp)
    bp_pad = jnp.zeros((1, PRED_PAD), jnp.float32).at[:, :2].set(bp)

    def _xspec(bb):
        return pl.BlockSpec((1, c, h * w), lambda j, bb=bb: (bb * t + j, 0, 0))

    feat_t, pred_t = pl.pallas_call(
        _fused_body,
        grid=(t,),
        in_specs=[
            _xspec(0), _xspec(1), _xspec(2), _xspec(3),
            pl.BlockSpec((K, K), lambda j: (0, 0)),
            pl.BlockSpec((K, c), lambda j: (0, 0)),
            pl.BlockSpec((1, K), lambda j: (0, 0)),
            pl.BlockSpec((h * w, HID), lambda j: (0, 0)),
            pl.BlockSpec((1, HID), lambda j: (0, 0)),
            pl.BlockSpec((HID, 3 * HID), lambda j: (0, 0)),
            pl.BlockSpec((1, 3 * HID), lambda j: (0, 0)),
            pl.BlockSpec((HID, 3 * HID), lambda j: (0, 0)),
            pl.BlockSpec((HID, PRED_PAD), lambda j: (0, 0)),
            pl.BlockSpec((1, PRED_PAD), lambda j: (0, 0)),
        ],
        out_specs=[
            pl.BlockSpec((1, BS * K, HID), lambda j: (j, 0, 0)),
            pl.BlockSpec((1, BS * K, PRED_PAD), lambda j: (j, 0, 0)),
        ],
        out_shape=[
            jax.ShapeDtypeStruct((t, BS * K, HID), jnp.float32),
            jax.ShapeDtypeStruct((t, BS * K, PRED_PAD), jnp.float32),
        ],
        scratch_shapes=[pltpu.VMEM((BS, K, HID), jnp.float32)],
        compiler_params=pltpu.CompilerParams(
            dimension_semantics=("arbitrary",)),
    )(xf, xf, xf, xf, A, Wconv, bconv.reshape(1, K), Wgb, bg.reshape(1, HID),
      Wzrn, bzrn, Uzrn, Wp_pad, bp_pad)

    feat = feat_t.reshape(t, b, K, HID).transpose(1, 0, 2, 3)
    pred = pred_t[..., :2].reshape(t, b, K, 2).transpose(1, 0, 2, 3)
    return pred, feat


# speedup vs baseline: 1.2041x; 1.0025x over previous
import jax
import jax.numpy as jnp
from jax.experimental import pallas as pl
from jax.experimental.pallas import tpu as pltpu

BS, T, C_IN = 4, 8, 256
K = 17
NODE_DIM = 1024
HID = 256
PRED_PAD = 128

_PREC = None


def _fused_body(x0_ref, x1_ref, x2_ref, x3_ref, A_ref, Wc_ref, bc_ref,
                Wg_ref, bg_ref, Wzrn_ref, bzrn_ref, Uzrn_ref, Wp_ref, bp_ref,
                feat_ref, pred_ref, h_scr):
    x_refs = (x0_ref, x1_ref, x2_ref, x3_ref)
    j = pl.program_id(0)

    @pl.when(j == 0)
    def _init():
        h_scr[...] = jnp.zeros_like(h_scr)

    AW = jnp.dot(A_ref[:], Wc_ref[:], precision=_PREC,
                 preferred_element_type=jnp.float32)
    ab = jnp.sum(A_ref[:] * bc_ref[:], axis=1, keepdims=True)

    hs = [h_scr[b] for b in range(BS)]
    hu = [jnp.dot(hs[b], Uzrn_ref[:], precision=_PREC,
                  preferred_element_type=jnp.float32) for b in range(BS)]
    g = [jnp.dot(AW, x_refs[b][0], precision=_PREC,
                 preferred_element_type=jnp.float32) + ab for b in range(BS)]
    gw = [jnp.maximum(
        jnp.dot(g[b].astype(jnp.bfloat16), Wg_ref[:], precision=_PREC,
                preferred_element_type=jnp.float32) + bg_ref[:], 0.0)
        for b in range(BS)]
    xp = [jnp.dot(gw[b], Wzrn_ref[:], precision=_PREC,
                  preferred_element_type=jnp.float32) + bzrn_ref[:]
          for b in range(BS)]
    for b in range(BS):
        z = jax.nn.sigmoid(xp[b][:, :HID] + hu[b][:, :HID])
        r = jax.nn.sigmoid(xp[b][:, HID:2 * HID] + hu[b][:, HID:2 * HID])
        n = jnp.tanh(xp[b][:, 2 * HID:] + r * hu[b][:, 2 * HID:])
        hn = hs[b] + z * (n - hs[b])
        h_scr[b] = hn
        feat_ref[0, b * K:(b + 1) * K] = hn
        pred_ref[0, b * K:(b + 1) * K] = jnp.dot(
            hn, Wp_ref[:], precision=_PREC,
            preferred_element_type=jnp.float32) + bp_ref[:]


def kernel(x, A, Wconv, bconv, Wg, bg, Wz, Uz, bz, Wr, Ur, br,
           Wn, Un, bn, Wp, bp):
    b, t, c, h, w = x.shape
    xf = x.reshape(b * t, c, h * w)
    Wgb = Wg.astype(jnp.bfloat16)

    Wzrn = jnp.concatenate([Wz, Wr, Wn], axis=1)
    bzrn = jnp.concatenate([bz, br, bn]).reshape(1, 3 * HID)
    Uzrn = jnp.concatenate([Uz, Ur, Un], axis=1)
    Wp_pad = jnp.zeros((HID, PRED_PAD), jnp.float32).at[:, :2].set(Wp)
    bp_pad = jnp.zeros((1, PRED_PAD), jnp.float32).at[:, :2].set(bp)

    def _xspec(bb):
        return pl.BlockSpec((1, c, h * w), lambda j, bb=bb: (bb * t + j, 0, 0))

    feat_t, pred_t = pl.pallas_call(
        _fused_body,
        grid=(t,),
        in_specs=[
            _xspec(0), _xspec(1), _xspec(2), _xspec(3),
            pl.BlockSpec((K, K), lambda j: (0, 0)),
            pl.BlockSpec((K, c), lambda j: (0, 0)),
            pl.BlockSpec((1, K), lambda j: (0, 0)),
            pl.BlockSpec((h * w, HID), lambda j: (0, 0)),
            pl.BlockSpec((1, HID), lambda j: (0, 0)),
            pl.BlockSpec((HID, 3 * HID), lambda j: (0, 0)),
            pl.BlockSpec((1, 3 * HID), lambda j: (0, 0)),
            pl.BlockSpec((HID, 3 * HID), lambda j: (0, 0)),
            pl.BlockSpec((HID, PRED_PAD), lambda j: (0, 0)),
            pl.BlockSpec((1, PRED_PAD), lambda j: (0, 0)),
        ],
        out_specs=[
            pl.BlockSpec((1, BS * K, HID), lambda j: (j, 0, 0)),
            pl.BlockSpec((1, BS * K, PRED_PAD), lambda j: (j, 0, 0)),
        ],
        out_shape=[
            jax.ShapeDtypeStruct((t, BS * K, HID), jnp.float32),
            jax.ShapeDtypeStruct((t, BS * K, PRED_PAD), jnp.float32),
        ],
        scratch_shapes=[pltpu.VMEM((BS, K, HID), jnp.float32)],
        compiler_params=pltpu.CompilerParams(
            dimension_semantics=("arbitrary",)),
    )(xf, xf, xf, xf, A, Wconv, bconv.reshape(1, K), Wgb, bg.reshape(1, HID),
      Wzrn, bzrn, Uzrn, Wp_pad, bp_pad)

    feat = feat_t.reshape(t, b, K, HID).transpose(1, 0, 2, 3)
    pred = pred_t[..., :2].reshape(t, b, K, 2).transpose(1, 0, 2, 3)
    return pred, feat
